# baseline (speedup 1.0000x reference)
import functools

import jax
import jax.numpy as jnp
from jax import lax
from jax.experimental import pallas as pl
from jax.experimental.pallas import tpu as pltpu
from jax.experimental.pallas import tpu_sc as plsc

_KEEP_RATIO = 0.7

_NC = 2
_NS = 16
_L = 16
_NW = _NC * _NS


def _sc_target_gather(x_flat_hbm, tgt_hbm, out_hbm, tgt_v, flat_v, tl_v, sem,
                      *, batch, ncls):
    b_per_w = batch // _NW
    nvec = b_per_w // _L
    nchunk = b_per_w // 128
    wid = lax.axis_index("s") * _NC + lax.axis_index("c")
    base = wid * b_per_w

    pltpu.sync_copy(tgt_hbm.at[pl.ds(base, b_per_w)], tgt_v)

    lane = lax.iota(jnp.int32, _L)
    for j in range(nvec):
        sl = pl.ds(j * _L, _L)
        flat_v[0, sl] = (base + j * _L + lane) * ncls + tgt_v[sl]

    for k in range(nchunk):
        pltpu.async_copy(
            x_flat_hbm.at[flat_v.at[0, pl.ds(k * 128, 128)]],
            tl_v.at[pl.ds(k * 128, 128)],
            sem,
        )
    for k in range(nchunk):
        pltpu.make_async_copy(
            x_flat_hbm.at[flat_v.at[0, pl.ds(k * 128, 128)]],
            tl_v.at[pl.ds(k * 128, 128)],
            sem,
        ).wait()

    pltpu.sync_copy(tl_v, out_hbm.at[pl.ds(base, b_per_w)])


def _tc_main(tl_ref, x_ref, out_ref, ce_ref, *, nblocks, rank):
    i = pl.program_id(0)
    x = x_ref[...]

    m = jnp.max(x, axis=1, keepdims=True)
    s = jnp.sum(jnp.exp(x - m), axis=1)
    lse = m[:, 0] + jnp.log(s)

    ce = (lse - tl_ref[0, 0, :]) + 0.0
    ce_ref[i, :] = ce

    @pl.when(i == nblocks - 1)
    def _select():
        int_min = jnp.int32(-2147483648)
        ce_all = ce_ref[...]
        u = lax.bitcast_convert_type(ce_all, jnp.int32)
        keys = jnp.where(u < 0, u ^ jnp.int32(0x7FFFFFFF), u)

        def body(b, p):
            bit = lax.shift_left(jnp.int32(1), jnp.int32(31) - b)
            cand = p | bit
            cand_cmp = cand ^ int_min
            cnt = jnp.sum((keys < cand_cmp).astype(jnp.int32))
            return jnp.where(cnt <= rank, cand, p)

        p = lax.fori_loop(0, 32, body, jnp.int32(0))
        thr_key = p ^ int_min

        mask = (keys >= thr_key).astype(jnp.float32)
        kept_sum = jnp.sum(ce_all * mask)
        kept_cnt = jnp.sum(mask)
        out_ref[0, 0] = kept_sum / (kept_cnt + 1e-8)


def kernel(inputs, targets):
    batch, ncls = inputs.shape
    block_rows = 2048
    nblocks = batch // block_rows
    num_keep = int(batch * _KEEP_RATIO)
    rank = batch - num_keep
    b_per_w = batch // _NW

    x_flat = inputs.reshape(batch * ncls)
    tgt = targets.astype(jnp.int32)

    sc_gather = pl.kernel(
        functools.partial(_sc_target_gather, batch=batch, ncls=ncls),
        out_type=jax.ShapeDtypeStruct((batch,), jnp.float32),
        mesh=plsc.VectorSubcoreMesh(core_axis_name="c", subcore_axis_name="s"),
        scratch_types=[
            pltpu.VMEM((b_per_w,), jnp.int32),
            pltpu.VMEM((1, b_per_w), jnp.int32),
            pltpu.VMEM((b_per_w,), jnp.float32),
            pltpu.SemaphoreType.DMA,
        ],
    )
    tl = sc_gather(x_flat, tgt)

    tl3 = tl.reshape(nblocks, 1, block_rows)

    out = pl.pallas_call(
        functools.partial(_tc_main, nblocks=nblocks, rank=rank),
        grid=(nblocks,),
        in_specs=[
            pl.BlockSpec((1, 1, block_rows), lambda i: (i, 0, 0)),
            pl.BlockSpec((block_rows, ncls), lambda i: (i, 0)),
        ],
        out_specs=pl.BlockSpec(memory_space=pltpu.SMEM),
        out_shape=jax.ShapeDtypeStruct((1, 1), jnp.float32),
        scratch_shapes=[pltpu.VMEM((nblocks, block_rows), jnp.float32)],
    )(tl3, inputs)
    return out[0, 0]


# speedup vs baseline: 1.0021x; 1.0021x over previous
import functools

import jax
import jax.numpy as jnp
from jax import lax
from jax.experimental import pallas as pl
from jax.experimental.pallas import tpu as pltpu
from jax.experimental.pallas import tpu_sc as plsc

_KEEP_RATIO = 0.7

_NC = 2
_NS = 16
_L = 16
_NW = _NC * _NS


def _sc_target_gather(x_flat_hbm, tgt_hbm, out_hbm, tgt_v, flat_v, tl_v, sem,
                      *, batch, ncls):
    b_per_w = batch // _NW
    nvec = b_per_w // _L
    nchunk = b_per_w // 128
    wid = lax.axis_index("s") * _NC + lax.axis_index("c")
    base = wid * b_per_w

    pltpu.sync_copy(tgt_hbm.at[pl.ds(base, b_per_w)], tgt_v)

    lane = lax.iota(jnp.int32, _L)
    for j in range(nvec):
        sl = pl.ds(j * _L, _L)
        flat_v[0, sl] = (base + j * _L + lane) * ncls + tgt_v[sl]

    for k in range(nchunk):
        pltpu.async_copy(
            x_flat_hbm.at[flat_v.at[0, pl.ds(k * 128, 128)]],
            tl_v.at[pl.ds(k * 128, 128)],
            sem,
        )
    for k in range(nchunk):
        pltpu.make_async_copy(
            x_flat_hbm.at[flat_v.at[0, pl.ds(k * 128, 128)]],
            tl_v.at[pl.ds(k * 128, 128)],
            sem,
        ).wait()

    pltpu.sync_copy(tl_v, out_hbm.at[pl.ds(base, b_per_w)])


def _tc_stream(x_ref, lse_ref, *, nblocks):
    x = x_ref[...]
    m = jnp.max(x, axis=1, keepdims=True)
    s = jnp.sum(jnp.exp(x - m), axis=1)
    lse_ref[0, 0, :] = m[:, 0] + jnp.log(s)


def _tc_select(lse_ref, tl_ref, out_ref, *, rank):
    ce_all = (lse_ref[...] - tl_ref[...]) + 0.0

    def _select():
        int_min = jnp.int32(-2147483648)
        u = lax.bitcast_convert_type(ce_all, jnp.int32)
        keys = jnp.where(u < 0, u ^ jnp.int32(0x7FFFFFFF), u)

        def body(b, p):
            bit = lax.shift_left(jnp.int32(1), jnp.int32(31) - b)
            cand = p | bit
            cand_cmp = cand ^ int_min
            cnt = jnp.sum((keys < cand_cmp).astype(jnp.int32))
            return jnp.where(cnt <= rank, cand, p)

        p = lax.fori_loop(0, 32, body, jnp.int32(0))
        thr_key = p ^ int_min

        mask = (keys >= thr_key).astype(jnp.float32)
        kept_sum = jnp.sum(ce_all * mask)
        kept_cnt = jnp.sum(mask)
        out_ref[0, 0] = kept_sum / (kept_cnt + 1e-8)

    _select()


def kernel(inputs, targets):
    batch, ncls = inputs.shape
    block_rows = 2048
    nblocks = batch // block_rows
    num_keep = int(batch * _KEEP_RATIO)
    rank = batch - num_keep
    b_per_w = batch // _NW

    x_flat = inputs.reshape(batch * ncls)
    tgt = targets.astype(jnp.int32)

    sc_gather = pl.kernel(
        functools.partial(_sc_target_gather, batch=batch, ncls=ncls),
        out_type=jax.ShapeDtypeStruct((batch,), jnp.float32),
        mesh=plsc.VectorSubcoreMesh(core_axis_name="c", subcore_axis_name="s"),
        scratch_types=[
            pltpu.VMEM((b_per_w,), jnp.int32),
            pltpu.VMEM((1, b_per_w), jnp.int32),
            pltpu.VMEM((b_per_w,), jnp.float32),
            pltpu.SemaphoreType.DMA,
        ],
    )
    tl = sc_gather(x_flat, tgt)

    lse = pl.pallas_call(
        functools.partial(_tc_stream, nblocks=nblocks),
        grid=(nblocks,),
        in_specs=[pl.BlockSpec((block_rows, ncls), lambda i: (i, 0))],
        out_specs=pl.BlockSpec((1, 1, block_rows), lambda i: (i, 0, 0)),
        out_shape=jax.ShapeDtypeStruct((nblocks, 1, block_rows), jnp.float32),
    )(inputs)

    out = pl.pallas_call(
        functools.partial(_tc_select, rank=rank),
        in_specs=[
            pl.BlockSpec((nblocks, 1, block_rows), lambda: (0, 0, 0)),
            pl.BlockSpec((nblocks, 1, block_rows), lambda: (0, 0, 0)),
        ],
        out_specs=pl.BlockSpec(memory_space=pltpu.SMEM),
        out_shape=jax.ShapeDtypeStruct((1, 1), jnp.float32),
    )(lse, tl.reshape(nblocks, 1, block_rows))
    return out[0, 0]


# speedup vs baseline: 2.0301x; 2.0260x over previous
import functools

import jax
import jax.numpy as jnp
from jax.experimental import pallas as pl
from jax.experimental.pallas import tpu as pltpu

_KEEP_RATIO = 0.7


def _ohem_kernel(targets_ref, x_ref, out_ref, ce_ref, *, nblocks, block_rows, rank):
    i = pl.program_id(0)
    x = x_ref[...]
    rows, ncls = x.shape

    m = jnp.max(x, axis=1, keepdims=True)
    s = jnp.sum(jnp.exp(x - m), axis=1)
    lse = m[:, 0] + jnp.log(s)

    t = targets_ref[0, 0, :]
    cls_iota = jax.lax.broadcasted_iota(jnp.int32, (rows, ncls), 1)
    tl = jnp.sum(jnp.where(cls_iota == t[:, None], x, 0.0), axis=1)

    ce = (lse - tl) + 0.0
    ce_ref[i, :] = ce

    @pl.when(i == nblocks - 1)
    def _select():
        int_min = jnp.int32(-2147483648)
        ce_all = ce_ref[...]
        u = jax.lax.bitcast_convert_type(ce_all, jnp.int32)
        keys = jnp.where(u < 0, u ^ jnp.int32(0x7FFFFFFF), u)

        p = jnp.int32(0)
        for sh in range(28, -1, -4):
            js = jnp.int32(0)
            for j in range(1, 16):
                jv = (j << sh) & 0xFFFFFFFF
                jv = jv - (1 << 32) if jv >= (1 << 31) else jv
                cand_cmp = (p | jnp.int32(jv)) ^ int_min
                cnt = jnp.sum((keys < cand_cmp).astype(jnp.int32))
                js += (cnt <= rank).astype(jnp.int32)
            p = p | jax.lax.shift_left(js, jnp.int32(sh))
        thr_key = p ^ int_min

        mask = (keys >= thr_key).astype(jnp.float32)
        kept_sum = jnp.sum(ce_all * mask)
        kept_cnt = jnp.sum(mask)
        out_ref[0, 0] = kept_sum / (kept_cnt + 1e-8)


def kernel(inputs, targets):
    batch, ncls = inputs.shape
    block_rows = 2048
    nblocks = batch // block_rows
    num_keep = int(batch * _KEEP_RATIO)
    rank = batch - num_keep

    targets3 = targets.astype(jnp.int32).reshape(nblocks, 1, block_rows)

    out = pl.pallas_call(
        functools.partial(
            _ohem_kernel, nblocks=nblocks, block_rows=block_rows, rank=rank
        ),
        grid=(nblocks,),
        in_specs=[
            pl.BlockSpec((1, 1, block_rows), lambda i: (i, 0, 0)),
            pl.BlockSpec((block_rows, ncls), lambda i: (i, 0)),
        ],
        out_specs=pl.BlockSpec(memory_space=pltpu.SMEM),
        out_shape=jax.ShapeDtypeStruct((1, 1), jnp.float32),
        scratch_shapes=[pltpu.VMEM((nblocks, block_rows), jnp.float32)],
    )(targets3, inputs)
    return out[0, 0]
